# baseline (speedup 1.0000x reference)
import functools

import jax
import jax.numpy as jnp
from jax import lax
from jax.experimental import pallas as pl
from jax.experimental.pallas import tpu as pltpu
from jax.experimental.pallas import tpu_sc as plsc

EMB_DIM = 64
NUM_WORKERS = 32
CHUNK = 512


def _emb_body(idx_hbm, tab_hbm, out_hbm, idx_v, rows_v, sem, *, rows_per_worker):
    wid = lax.axis_index("s") * 2 + lax.axis_index("c")
    base = wid * rows_per_worker
    nchunks = rows_per_worker // CHUNK

    def body(i, carry):
        off = base + i * CHUNK
        pltpu.sync_copy(idx_hbm.at[pl.ds(off, CHUNK)], idx_v)
        pltpu.async_copy(tab_hbm.at[idx_v], rows_v, sem).wait()
        pltpu.sync_copy(rows_v, out_hbm.at[pl.ds(off, CHUNK)])
        return carry

    lax.fori_loop(0, nchunks, body, 0)


def kernel(x, table):
    b, h = x.shape
    n = b * h
    rows_per_worker = n // NUM_WORKERS
    idx = x.reshape(n).astype(jnp.int32)

    mesh = plsc.VectorSubcoreMesh(core_axis_name="c", subcore_axis_name="s")
    emb = functools.partial(
        pl.kernel,
        mesh=mesh,
        out_type=jax.ShapeDtypeStruct((n, EMB_DIM), jnp.float32),
        scratch_types=[
            pltpu.VMEM((CHUNK,), jnp.int32),
            pltpu.VMEM((CHUNK, EMB_DIM), jnp.float32),
            pltpu.SemaphoreType.DMA,
        ],
        compiler_params=pltpu.CompilerParams(use_tc_tiling_on_sc=False),
    )(functools.partial(_emb_body, rows_per_worker=rows_per_worker))

    out = emb(idx, table)
    return out.reshape(b, h, EMB_DIM)


# speedup vs baseline: 1.0416x; 1.0416x over previous
import functools

import jax
import jax.numpy as jnp
from jax import lax
from jax.experimental import pallas as pl
from jax.experimental.pallas import tpu as pltpu
from jax.experimental.pallas import tpu_sc as plsc

EMB_DIM = 64
NUM_WORKERS = 32
CHUNK = 512


def _emb_body(idx_hbm, tab_hbm, out_hbm,
              idx0, idx1, rows0, rows1,
              si0, si1, sg0, sg1, so0, so1, *, rows_per_worker):
    wid = lax.axis_index("s") * 2 + lax.axis_index("c")
    base = wid * rows_per_worker
    n = rows_per_worker // CHUNK
    idxv = (idx0, idx1)
    rowsv = (rows0, rows1)
    si = (si0, si1)
    sg = (sg0, sg1)
    so = (so0, so1)

    def idx_slice(i):
        return idx_hbm.at[pl.ds(base + i * CHUNK, CHUNK)]

    def out_slice(i):
        return out_hbm.at[pl.ds(base + i * CHUNK, CHUNK)]

    def start_idx(i, b):
        pltpu.async_copy(idx_slice(i), idxv[b], si[b])

    def wait_idx(b):
        pltpu.make_async_copy(idx_slice(0), idxv[b], si[b]).wait()

    def start_gather(b):
        pltpu.async_copy(tab_hbm.at[idxv[b]], rowsv[b], sg[b])

    def wait_gather(b):
        pltpu.make_async_copy(tab_hbm.at[idxv[b]], rowsv[b], sg[b]).wait()

    def start_wb(i, b):
        pltpu.async_copy(rowsv[b], out_slice(i), so[b])

    def wait_wb(b):
        pltpu.make_async_copy(rowsv[b], out_slice(0), so[b]).wait()

    pltpu.sync_copy(idx_slice(0), idxv[0])
    start_gather(0)
    start_idx(1, 1)

    wait_gather(0)
    start_wb(0, 0)
    wait_idx(1)
    start_gather(1)
    start_idx(2, 0)

    wait_gather(1)
    start_wb(1, 1)
    wait_idx(0)
    wait_wb(0)
    start_gather(0)
    start_idx(3, 1)

    def body(j, carry):
        for b in (0, 1):
            i = 2 * j + b
            wait_gather(b)
            start_wb(i, b)
            wait_idx(b ^ 1)
            wait_wb(b ^ 1)
            start_gather(b ^ 1)
            start_idx(i + 2, b)
        return carry

    lax.fori_loop(1, n // 2 - 1, body, 0)

    wait_gather(0)
    start_wb(n - 2, 0)
    wait_idx(1)
    wait_wb(1)
    start_gather(1)

    wait_gather(1)
    start_wb(n - 1, 1)
    wait_wb(0)
    wait_wb(1)


def kernel(x, table):
    b, h = x.shape
    n = b * h
    rows_per_worker = n // NUM_WORKERS
    idx = x.reshape(n).astype(jnp.int32)

    mesh = plsc.VectorSubcoreMesh(core_axis_name="c", subcore_axis_name="s")
    emb = functools.partial(
        pl.kernel,
        mesh=mesh,
        out_type=jax.ShapeDtypeStruct((n, EMB_DIM), jnp.float32),
        scratch_types=[
            pltpu.VMEM((CHUNK,), jnp.int32),
            pltpu.VMEM((CHUNK,), jnp.int32),
            pltpu.VMEM((CHUNK, EMB_DIM), jnp.float32),
            pltpu.VMEM((CHUNK, EMB_DIM), jnp.float32),
            pltpu.SemaphoreType.DMA,
            pltpu.SemaphoreType.DMA,
            pltpu.SemaphoreType.DMA,
            pltpu.SemaphoreType.DMA,
            pltpu.SemaphoreType.DMA,
            pltpu.SemaphoreType.DMA,
        ],
        compiler_params=pltpu.CompilerParams(use_tc_tiling_on_sc=False),
    )(functools.partial(_emb_body, rows_per_worker=rows_per_worker))

    out = emb(idx, table)
    return out.reshape(b, h, EMB_DIM)


# speedup vs baseline: 1.2749x; 1.2239x over previous
import functools

import jax
import jax.numpy as jnp
from jax import lax
from jax.experimental import pallas as pl
from jax.experimental.pallas import tpu as pltpu
from jax.experimental.pallas import tpu_sc as plsc

EMB_DIM = 64
PAD_DIM = 128
NUM_WORKERS = 32
CHUNK = 400


def _emb_body(idx_hbm, tab_hbm, out_hbm,
              idx0, idx1, rows0, rows1,
              si0, si1, sg0, sg1, so0, so1, *, rows_per_worker):
    wid = lax.axis_index("s") * 2 + lax.axis_index("c")
    base = wid * rows_per_worker
    n = rows_per_worker // CHUNK
    idxv = (idx0, idx1)
    rowsv = (rows0, rows1)
    si = (si0, si1)
    sg = (sg0, sg1)
    so = (so0, so1)

    def idx_slice(i):
        return idx_hbm.at[pl.ds(base + i * CHUNK, CHUNK)]

    def out_slice(i):
        return out_hbm.at[pl.ds(base + i * CHUNK, CHUNK)]

    def start_idx(i, b):
        pltpu.async_copy(idx_slice(i), idxv[b], si[b])

    def wait_idx(b):
        pltpu.make_async_copy(idx_slice(0), idxv[b], si[b]).wait()

    def start_gather(b):
        pltpu.async_copy(tab_hbm.at[idxv[b]], rowsv[b], sg[b])

    def wait_gather(b):
        pltpu.make_async_copy(tab_hbm.at[idxv[b]], rowsv[b], sg[b]).wait()

    def start_wb(i, b):
        pltpu.async_copy(rowsv[b], out_slice(i), so[b])

    def wait_wb(b):
        pltpu.make_async_copy(rowsv[b], out_slice(0), so[b]).wait()

    pltpu.sync_copy(idx_slice(0), idxv[0])
    start_gather(0)
    start_idx(1, 1)

    wait_gather(0)
    start_wb(0, 0)
    wait_idx(1)
    start_gather(1)
    start_idx(2, 0)

    wait_gather(1)
    start_wb(1, 1)
    wait_idx(0)
    wait_wb(0)
    start_gather(0)
    start_idx(3, 1)

    def body(j, carry):
        for b in (0, 1):
            i = 2 * j + b
            wait_gather(b)
            start_wb(i, b)
            wait_idx(b ^ 1)
            wait_wb(b ^ 1)
            start_gather(b ^ 1)
            start_idx(i + 2, b)
        return carry

    lax.fori_loop(1, n // 2 - 1, body, 0)

    wait_gather(0)
    start_wb(n - 2, 0)
    wait_idx(1)
    wait_wb(1)
    start_gather(1)

    wait_gather(1)
    start_wb(n - 1, 1)
    wait_wb(0)
    wait_wb(1)


def kernel(x, table):
    b, h = x.shape
    n = b * h
    rows_per_worker = n // NUM_WORKERS
    idx = x.reshape(n).astype(jnp.int32)
    tab_pad = jnp.pad(table, ((0, 0), (0, PAD_DIM - EMB_DIM)))

    mesh = plsc.VectorSubcoreMesh(core_axis_name="c", subcore_axis_name="s")
    emb = functools.partial(
        pl.kernel,
        mesh=mesh,
        out_type=jax.ShapeDtypeStruct((n, PAD_DIM), jnp.float32),
        scratch_types=[
            pltpu.VMEM((CHUNK,), jnp.int32),
            pltpu.VMEM((CHUNK,), jnp.int32),
            pltpu.VMEM((CHUNK, PAD_DIM), jnp.float32),
            pltpu.VMEM((CHUNK, PAD_DIM), jnp.float32),
            pltpu.SemaphoreType.DMA,
            pltpu.SemaphoreType.DMA,
            pltpu.SemaphoreType.DMA,
            pltpu.SemaphoreType.DMA,
            pltpu.SemaphoreType.DMA,
            pltpu.SemaphoreType.DMA,
        ],
    )(functools.partial(_emb_body, rows_per_worker=rows_per_worker))

    out = emb(idx, tab_pad)
    return out[:, :EMB_DIM].reshape(b, h, EMB_DIM)
